# baseline (speedup 1.0000x reference)
import functools

import jax
import jax.numpy as jnp
from jax import lax
from jax.experimental import pallas as pl
from jax.experimental.pallas import tpu as pltpu
from jax.experimental.pallas import tpu_sc as plsc

EMB_SIZE = 1000000
HALF = EMB_SIZE // 2
EMB_DIM = 32
LANES = 16
NUM_CORES = 2
NUM_SUBCORES = 16
NUM_WORKERS = NUM_CORES * NUM_SUBCORES
BATCH = 16384
BPW = BATCH // NUM_WORKERS
GCHUNK = 128


def _sc_body(p1_hbm, p2_hbm, wa_hbm, wb_hbm, out_hbm, idx1_v, idx2_v,
             ia_v, ib_v, rows_v, out_v, sem):
  wid = lax.axis_index("s") * NUM_CORES + lax.axis_index("c")
  base = wid * BPW

  pltpu.sync_copy(p1_hbm.at[pl.ds(base, BPW)], idx1_v)
  pltpu.sync_copy(p2_hbm.at[pl.ds(base, BPW)], idx2_v)

  def split_body(j, carry):
    sl = pl.ds(j * LANES, LANES)
    spread = lax.broadcasted_iota(jnp.int32, (LANES,), 0) * 1024 + j
    for idx_v, off in ((idx1_v, 0), (idx2_v, BPW)):
      iv = idx_v[sl]
      in_a = iv < HALF
      ia_v[pl.ds(off + j * LANES, LANES)] = jnp.where(in_a, iv, spread)
      ib_v[pl.ds(off + j * LANES, LANES)] = jnp.where(in_a, spread, iv - HALF)
    return carry

  lax.fori_loop(0, BPW // LANES, split_body, 0)

  for half in range(2):
    copies = []
    for j in range(half * 4, half * 4 + 4):
      isl = pl.ds(j * GCHUNK, GCHUNK)
      copies.append(pltpu.async_copy(
          wa_hbm.at[ia_v.at[isl]], rows_v.at[pl.ds(2 * j * GCHUNK, GCHUNK)],
          sem))
      copies.append(pltpu.async_copy(
          wb_hbm.at[ib_v.at[isl]],
          rows_v.at[pl.ds((2 * j + 1) * GCHUNK, GCHUNK)], sem))
    for cp in copies:
      cp.wait()

  def chunk_body(c, carry):
    sl = pl.ds(c * LANES, LANES)
    i1 = idx1_v[sl]
    i2 = idx2_v[sl]
    m1 = i1 < HALF
    m2 = i2 < HALF
    lane = lax.broadcasted_iota(jnp.int32, (LANES,), 0)
    pos = c * LANES + lane
    chunk_id = pos // GCHUNK
    off_in = pos % GCHUNK
    r1 = jnp.where(m1, 2 * chunk_id * GCHUNK,
                   (2 * chunk_id + 1) * GCHUNK) + off_in
    p2base = 2 * BPW
    r2 = jnp.where(m2, p2base + 2 * chunk_id * GCHUNK,
                   p2base + (2 * chunk_id + 1) * GCHUNK) + off_in
    acc = jnp.zeros((LANES,), jnp.float32)
    for d in range(EMB_DIM):
      col = jnp.full((LANES,), d, jnp.int32)
      a = plsc.load_gather(rows_v, [r1, col])
      b = plsc.load_gather(rows_v, [r2, col])
      acc = acc + a * b
    out_v[sl] = acc
    return carry

  lax.fori_loop(0, BPW // LANES, chunk_body, 0)

  pltpu.sync_copy(out_v, out_hbm.at[pl.ds(base, BPW)])


@jax.jit
def _mf_dot(product1, product2, wa, wb):
  mesh = plsc.VectorSubcoreMesh(core_axis_name="c", subcore_axis_name="s")
  f = pl.kernel(
      _sc_body,
      out_type=jax.ShapeDtypeStruct((BATCH,), jnp.float32),
      mesh=mesh,
      scratch_types=[
          pltpu.VMEM((BPW,), jnp.int32),
          pltpu.VMEM((BPW,), jnp.int32),
          pltpu.VMEM((2 * BPW,), jnp.int32),
          pltpu.VMEM((2 * BPW,), jnp.int32),
          pltpu.VMEM((4 * BPW, EMB_DIM), jnp.float32),
          pltpu.VMEM((BPW,), jnp.float32),
          pltpu.SemaphoreType.DMA,
      ],
      compiler_params=pltpu.CompilerParams(needs_layout_passes=False,
                                           use_tc_tiling_on_sc=False),
  )
  return f(product1, product2, wa, wb)


def kernel(product1, product2, embedding_weight):
  wa = embedding_weight[:HALF]
  wb = embedding_weight[HALF:]
  return _mf_dot(product1.astype(jnp.int32), product2.astype(jnp.int32),
                 wa, wb)


# speedup vs baseline: 1.4076x; 1.4076x over previous
import functools

import jax
import jax.numpy as jnp
from jax import lax
from jax.experimental import pallas as pl
from jax.experimental.pallas import tpu as pltpu
from jax.experimental.pallas import tpu_sc as plsc

EMB_DIM = 32
LANES = 16
NUM_CORES = 2
NUM_SUBCORES = 16
NUM_WORKERS = NUM_CORES * NUM_SUBCORES
BATCH = 16384
BPW = BATCH // NUM_WORKERS
GCHUNK = 128


def _sc_body(p1_hbm, p2_hbm, w_hbm, out_hbm, idx1_v, idx2_v, rows1_v,
             rows2_v, out_v, sem):
  wid = lax.axis_index("s") * NUM_CORES + lax.axis_index("c")
  base = wid * BPW

  pltpu.sync_copy(p1_hbm.at[pl.ds(base, BPW)], idx1_v)
  pltpu.sync_copy(p2_hbm.at[pl.ds(base, BPW)], idx2_v)

  copies = []
  for j in range(BPW // GCHUNK):
    sl = pl.ds(j * GCHUNK, GCHUNK)
    copies.append(pltpu.async_copy(w_hbm.at[idx1_v.at[sl]], rows1_v.at[sl], sem))
    copies.append(pltpu.async_copy(w_hbm.at[idx2_v.at[sl]], rows2_v.at[sl], sem))
  for cp in copies:
    cp.wait()

  def chunk_body(c, carry):
    row_ids = lax.broadcasted_iota(jnp.int32, (LANES,), 0) + c * LANES
    acc = jnp.zeros((LANES,), jnp.float32)
    for d in range(EMB_DIM):
      col = jnp.full((LANES,), d, jnp.int32)
      a = plsc.load_gather(rows1_v, [row_ids, col])
      b = plsc.load_gather(rows2_v, [row_ids, col])
      acc = acc + a * b
    out_v[pl.ds(c * LANES, LANES)] = acc
    return carry

  lax.fori_loop(0, BPW // LANES, chunk_body, 0)

  pltpu.sync_copy(out_v, out_hbm.at[pl.ds(base, BPW)])


@jax.jit
def _mf_dot(product1, product2, embedding_weight):
  mesh = plsc.VectorSubcoreMesh(core_axis_name="c", subcore_axis_name="s")
  f = pl.kernel(
      _sc_body,
      out_type=jax.ShapeDtypeStruct((BATCH,), jnp.float32),
      mesh=mesh,
      scratch_types=[
          pltpu.VMEM((BPW,), jnp.int32),
          pltpu.VMEM((BPW,), jnp.int32),
          pltpu.VMEM((BPW, EMB_DIM), jnp.float32),
          pltpu.VMEM((BPW, EMB_DIM), jnp.float32),
          pltpu.VMEM((BPW,), jnp.float32),
          pltpu.SemaphoreType.DMA,
      ],
      compiler_params=pltpu.CompilerParams(needs_layout_passes=False,
                                           use_tc_tiling_on_sc=False),
  )
  return f(product1, product2, embedding_weight)


def kernel(product1, product2, embedding_weight):
  return _mf_dot(product1.astype(jnp.int32), product2.astype(jnp.int32),
                 embedding_weight)
